# baseline (speedup 1.0000x reference)
import functools

import jax
import jax.numpy as jnp
from jax import lax
from jax.experimental import pallas as pl
from jax.experimental.pallas import tpu as pltpu
from jax.experimental.pallas import tpu_sc as plsc

B = 16384
HIST = 50
PD = 64
HID = 128
NC = 2
NS = 16
NW = NC * NS
S = B // NW
SPS = 2
NBUF = 6
NFLY = 4
NP = S // SPS
L = 16
HP = HIST // 2
HPP = 32


def _sc_gather(hist2, pids, table):
    mesh = plsc.VectorSubcoreMesh(core_axis_name="c", subcore_axis_name="s")

    @functools.partial(
        pl.kernel,
        out_type=(jax.ShapeDtypeStruct((B * 2 * HPP, PD), jnp.float32),
                  jax.ShapeDtypeStruct((B, PD), jnp.float32)),
        mesh=mesh,
        compiler_params=pltpu.CompilerParams(needs_layout_passes=False,
                                             use_tc_tiling_on_sc=False),
        scratch_types=[
            pltpu.VMEM((NP, SPS * HIST), jnp.int32),
            pltpu.VMEM((S,), jnp.int32),
            pltpu.VMEM((128, PD), jnp.float32),
            pltpu.VMEM((NBUF, SPS * HIST, PD), jnp.float32),
            pltpu.SemaphoreType.DMA((NBUF,)),
            pltpu.SemaphoreType.DMA((NBUF,)),
            pltpu.SemaphoreType.DMA,
        ],
    )
    def k(hist_hbm, pid_hbm, table_hbm, histout_hbm, prodout_hbm,
          hidx_v, pidx_v, prow_v, ring_v, gsems, wsems, gsem):
        cid = lax.axis_index("c")
        sid = lax.axis_index("s")
        wid = cid * NS + sid
        base = wid * S

        pltpu.sync_copy(hist_hbm.at[pl.ds(wid * NP, NP)], hidx_v)
        pltpu.sync_copy(pid_hbm.at[pl.ds(base, S)], pidx_v)

        for kk in range(S // 128):
            pltpu.async_copy(
                table_hbm.at[pidx_v.at[pl.ds(kk * 128, 128)]], prow_v, gsem
            ).wait()
            pltpu.sync_copy(prow_v, prodout_hbm.at[pl.ds(base + kk * 128, 128)])

        def gcopy(p, slot):
            return pltpu.make_async_copy(
                table_hbm.at[hidx_v.at[p]], ring_v.at[slot], gsems.at[slot])

        def wcopy(p, sloc, slot):
            i = base + p * SPS + sloc
            return pltpu.make_async_copy(
                ring_v.at[slot, pl.ds(sloc * HIST, HIST)],
                histout_hbm.at[pl.ds(i * 2 * HPP, HIST)],
                wsems.at[slot])

        for j in range(NFLY):
            gcopy(j, j).start()

        @pl.loop(0, NP)
        def _(p):
            u = lax.rem(p, NBUF)
            gcopy(p, u).wait()
            wcopy(p, 0, u).start()
            wcopy(p, 1, u).start()
            nxt = p + NFLY
            v = lax.rem(nxt, NBUF)

            @pl.when(jnp.logical_and(nxt < NP, nxt >= NBUF))
            def _():
                wcopy(nxt, 0, v).wait()
                wcopy(nxt, 1, v).wait()
                gcopy(nxt, v).start()

            @pl.when(jnp.logical_and(nxt < NP, nxt < NBUF))
            def _():
                gcopy(nxt, v).start()

        for u in range(NBUF):
            wcopy(0, 0, u).wait()
            wcopy(0, 1, u).wait()

    return k(hist2, pids, table)


def _tc_attn_mlp(histp, prod, w2c, b_s, w1a, w1b, b1, w2, b2):
    BS = 256

    def body(h_ref, q_ref, w2c_ref, bs_ref, w1a_ref, w1b_ref, b1_ref,
             w2_ref, b2_ref, o_ref):
        h2d = h_ref[...]
        s2 = jnp.dot(h2d, w2c_ref[...],
                     preferred_element_type=jnp.float32)
        rows = lax.broadcasted_iota(jnp.int32, (BS * HPP, 2), 0)
        valid = lax.rem(rows, HPP) < HP
        e2 = jnp.where(valid, jnp.exp(jnp.tanh(s2 + bs_ref[...])), 0.0)
        e_e = jnp.broadcast_to(e2[:, 0:1], (BS * HPP, PD))
        e_o = jnp.broadcast_to(e2[:, 1:2], (BS * HPP, PD))
        eb = jnp.concatenate([e_e, e_o], axis=1)
        wh = (h2d * eb).reshape(BS, HPP, 2 * PD)
        ssum = jnp.sum(wh, axis=1)
        den = jnp.sum(jnp.sum(e2.reshape(BS, HPP, 2), axis=1),
                      axis=1, keepdims=True)
        pooled = (ssum[:, :PD] + ssum[:, PD:]) / den
        x = jnp.dot(pooled, w1a_ref[...], preferred_element_type=jnp.float32)
        x = x + jnp.dot(q_ref[...], w1b_ref[...],
                        preferred_element_type=jnp.float32)
        x = jnp.maximum(x + b1_ref[...], 0.0)
        y = jnp.dot(x, w2_ref[...], preferred_element_type=jnp.float32)
        o_ref[...] = jax.nn.sigmoid(y + b2_ref[...])

    return pl.pallas_call(
        body,
        grid=(B // BS,),
        in_specs=[
            pl.BlockSpec((BS * HPP, 2 * PD), lambda i: (i, 0)),
            pl.BlockSpec((BS, PD), lambda i: (i, 0)),
            pl.BlockSpec((2 * PD, 2), lambda i: (0, 0)),
            pl.BlockSpec((1, 1), lambda i: (0, 0)),
            pl.BlockSpec((PD, HID), lambda i: (0, 0)),
            pl.BlockSpec((PD, HID), lambda i: (0, 0)),
            pl.BlockSpec((1, HID), lambda i: (0, 0)),
            pl.BlockSpec((HID, 1), lambda i: (0, 0)),
            pl.BlockSpec((1, 1), lambda i: (0, 0)),
        ],
        out_specs=pl.BlockSpec((BS, 1), lambda i: (i, 0)),
        out_shape=jax.ShapeDtypeStruct((B, 1), jnp.float32),
    )(histp, prod, w2c, b_s, w1a, w1b, b1, w2, b2)


def kernel(user_ids, product_ids, user_history, user_table, prod_table,
           attn_W, attn_b, mlp1_W, mlp1_b, mlp2_W, mlp2_b):
    hist2 = user_history.astype(jnp.int32).reshape(B // SPS, SPS * HIST)
    pids = product_ids.astype(jnp.int32)
    packed = jax.lax.optimization_barrier(
        jnp.concatenate([prod_table[0::2], prod_table[1::2]], axis=1))
    table_rm = packed.reshape(1000000, PD)
    hist_rows, prodrows = _sc_gather(hist2, pids, table_rm)
    histp = hist_rows.reshape(B * HPP, 2 * PD)
    wcol = attn_W.reshape(PD)
    zz = jnp.zeros((PD,), jnp.float32)
    w2c = jnp.stack([jnp.concatenate([wcol, zz]),
                     jnp.concatenate([zz, wcol])], axis=1)
    out = _tc_attn_mlp(histp, prodrows,
                       w2c,
                       attn_b.reshape(1, 1),
                       mlp1_W[:PD], mlp1_W[PD:],
                       mlp1_b.reshape(1, HID), mlp2_W, mlp2_b.reshape(1, 1))
    return out


# speedup vs baseline: 9.4044x; 9.4044x over previous
import functools

import jax
import jax.numpy as jnp
from jax import lax
from jax.experimental import pallas as pl
from jax.experimental.pallas import tpu as pltpu
from jax.experimental.pallas import tpu_sc as plsc

B = 16384
HIST = 50
PD = 64
HID = 128
NC = 2
NS = 16
NW = NC * NS
S = B // NW
SPS = 2
NBUF = 6
NFLY = 4
NP = S // SPS
L = 16
HP = HIST // 2
HPP = 32


def _sc_gather(hist2, pids, table, nb):
    S = nb // NW
    NP = S // SPS
    mesh = plsc.VectorSubcoreMesh(core_axis_name="c", subcore_axis_name="s")

    @functools.partial(
        pl.kernel,
        out_type=(jax.ShapeDtypeStruct((nb * 2 * HPP, PD), jnp.float32),
                  jax.ShapeDtypeStruct((nb, PD), jnp.float32)),
        mesh=mesh,
        compiler_params=pltpu.CompilerParams(needs_layout_passes=False,
                                             use_tc_tiling_on_sc=False),
        scratch_types=[
            pltpu.VMEM((NP, SPS * HIST), jnp.int32),
            pltpu.VMEM((S,), jnp.int32),
            pltpu.VMEM((128, PD), jnp.float32),
            pltpu.VMEM((NBUF, SPS * HIST, PD), jnp.float32),
            pltpu.SemaphoreType.DMA((NBUF,)),
            pltpu.SemaphoreType.DMA((NBUF,)),
            pltpu.SemaphoreType.DMA,
        ],
    )
    def k(hist_hbm, pid_hbm, table_hbm, histout_hbm, prodout_hbm,
          hidx_v, pidx_v, prow_v, ring_v, gsems, wsems, gsem):
        cid = lax.axis_index("c")
        sid = lax.axis_index("s")
        wid = cid * NS + sid
        base = wid * S

        pltpu.sync_copy(hist_hbm.at[pl.ds(wid * NP, NP)], hidx_v)
        pltpu.sync_copy(pid_hbm.at[pl.ds(base, S)], pidx_v)

        for kk in range(S // 128):
            pltpu.async_copy(
                table_hbm.at[pidx_v.at[pl.ds(kk * 128, 128)]], prow_v, gsem
            ).wait()
            pltpu.sync_copy(prow_v, prodout_hbm.at[pl.ds(base + kk * 128, 128)])

        def gcopy(p, slot):
            return pltpu.make_async_copy(
                table_hbm.at[hidx_v.at[p]], ring_v.at[slot], gsems.at[slot])

        def wcopy(p, sloc, slot):
            i = base + p * SPS + sloc
            return pltpu.make_async_copy(
                ring_v.at[slot, pl.ds(sloc * HIST, HIST)],
                histout_hbm.at[pl.ds(i * 2 * HPP, HIST)],
                wsems.at[slot])

        for j in range(NFLY):
            gcopy(j, j).start()

        @pl.loop(0, NP)
        def _(p):
            u = lax.rem(p, NBUF)
            gcopy(p, u).wait()
            wcopy(p, 0, u).start()
            wcopy(p, 1, u).start()
            nxt = p + NFLY
            v = lax.rem(nxt, NBUF)

            @pl.when(jnp.logical_and(nxt < NP, nxt >= NBUF))
            def _():
                wcopy(nxt, 0, v).wait()
                wcopy(nxt, 1, v).wait()
                gcopy(nxt, v).start()

            @pl.when(jnp.logical_and(nxt < NP, nxt < NBUF))
            def _():
                gcopy(nxt, v).start()

        for u in range(NBUF):
            wcopy(0, 0, u).wait()
            wcopy(0, 1, u).wait()

    return k(hist2, pids, table)


def _tc_attn_mlp(histp, prod, w2c, b_s, w1a, w1b, b1, w2, b2, nb):
    BS = 256

    def body(h_ref, q_ref, w2c_ref, bs_ref, w1a_ref, w1b_ref, b1_ref,
             w2_ref, b2_ref, o_ref):
        h2d = h_ref[...]
        s2 = jnp.dot(h2d, w2c_ref[...],
                     preferred_element_type=jnp.float32)
        rows = lax.broadcasted_iota(jnp.int32, (BS * HPP, 2), 0)
        valid = lax.rem(rows, HPP) < HP
        e2 = jnp.where(valid, jnp.exp(jnp.tanh(s2 + bs_ref[...])), 0.0)
        e_e = jnp.broadcast_to(e2[:, 0:1], (BS * HPP, PD))
        e_o = jnp.broadcast_to(e2[:, 1:2], (BS * HPP, PD))
        eb = jnp.concatenate([e_e, e_o], axis=1)
        wh = (h2d * eb).reshape(BS, HPP, 2 * PD)
        ssum = jnp.sum(wh, axis=1)
        den = jnp.sum(jnp.sum(e2.reshape(BS, HPP, 2), axis=1),
                      axis=1, keepdims=True)
        pooled = (ssum[:, :PD] + ssum[:, PD:]) / den
        x = jnp.dot(pooled, w1a_ref[...], preferred_element_type=jnp.float32)
        x = x + jnp.dot(q_ref[...], w1b_ref[...],
                        preferred_element_type=jnp.float32)
        x = jnp.maximum(x + b1_ref[...], 0.0)
        y = jnp.dot(x, w2_ref[...], preferred_element_type=jnp.float32)
        o_ref[...] = jax.nn.sigmoid(y + b2_ref[...])

    return pl.pallas_call(
        body,
        grid=(nb // BS,),
        in_specs=[
            pl.BlockSpec((BS * HPP, 2 * PD), lambda i: (i, 0)),
            pl.BlockSpec((BS, PD), lambda i: (i, 0)),
            pl.BlockSpec((2 * PD, 2), lambda i: (0, 0)),
            pl.BlockSpec((1, 1), lambda i: (0, 0)),
            pl.BlockSpec((PD, HID), lambda i: (0, 0)),
            pl.BlockSpec((PD, HID), lambda i: (0, 0)),
            pl.BlockSpec((1, HID), lambda i: (0, 0)),
            pl.BlockSpec((HID, 1), lambda i: (0, 0)),
            pl.BlockSpec((1, 1), lambda i: (0, 0)),
        ],
        out_specs=pl.BlockSpec((BS, 1), lambda i: (i, 0)),
        out_shape=jax.ShapeDtypeStruct((nb, 1), jnp.float32),
    )(histp, prod, w2c, b_s, w1a, w1b, b1, w2, b2)


def kernel(user_ids, product_ids, user_history, user_table, prod_table,
           attn_W, attn_b, mlp1_W, mlp1_b, mlp2_W, mlp2_b):
    hist2 = user_history.astype(jnp.int32).reshape(B // SPS, SPS * HIST)
    pids = product_ids.astype(jnp.int32)
    wcol = attn_W.reshape(PD)
    zz = jnp.zeros((PD,), jnp.float32)
    w2c = jnp.stack([jnp.concatenate([wcol, zz]),
                     jnp.concatenate([zz, wcol])], axis=1)
    NCH = 2
    nb = B // NCH
    outs = []
    for c in range(NCH):
        h2c = lax.slice_in_dim(hist2, c * (nb // SPS), (c + 1) * (nb // SPS))
        pc = lax.slice_in_dim(pids, c * nb, (c + 1) * nb)
        hist_rows, prodrows = _sc_gather(h2c, pc, prod_table, nb)
        histp = hist_rows.reshape(nb * HPP, 2 * PD)
        outs.append(_tc_attn_mlp(histp, prodrows,
                                 w2c,
                                 attn_b.reshape(1, 1),
                                 mlp1_W[:PD], mlp1_W[PD:],
                                 mlp1_b.reshape(1, HID), mlp2_W,
                                 mlp2_b.reshape(1, 1), nb))
    return jnp.concatenate(outs, axis=0)


# speedup vs baseline: 9.5151x; 1.0118x over previous
import functools

import jax
import jax.numpy as jnp
from jax import lax
from jax.experimental import pallas as pl
from jax.experimental.pallas import tpu as pltpu
from jax.experimental.pallas import tpu_sc as plsc

B = 16384
HIST = 50
PD = 64
HID = 128
NC = 2
NS = 16
NW = NC * NS
S = B // NW
SPS = 2
NBUF = 6
NFLY = 4
NP = S // SPS
L = 16
HP = HIST // 2
HPP = 32


def _sc_gather(hist2, pids, table, nb):
    S = nb // NW
    NP = S // SPS
    mesh = plsc.VectorSubcoreMesh(core_axis_name="c", subcore_axis_name="s")

    @functools.partial(
        pl.kernel,
        out_type=(jax.ShapeDtypeStruct((nb * 2 * HPP, PD), jnp.float32),
                  jax.ShapeDtypeStruct((nb, PD), jnp.float32)),
        mesh=mesh,
        compiler_params=pltpu.CompilerParams(needs_layout_passes=False,
                                             use_tc_tiling_on_sc=False),
        scratch_types=[
            pltpu.VMEM((NP, SPS * HIST), jnp.int32),
            pltpu.VMEM((S,), jnp.int32),
            pltpu.VMEM((128, PD), jnp.float32),
            pltpu.VMEM((NBUF, SPS * HIST, PD), jnp.float32),
            pltpu.SemaphoreType.DMA((NBUF,)),
            pltpu.SemaphoreType.DMA((NBUF,)),
            pltpu.SemaphoreType.DMA,
        ],
    )
    def k(hist_hbm, pid_hbm, table_hbm, histout_hbm, prodout_hbm,
          hidx_v, pidx_v, prow_v, ring_v, gsems, wsems, gsem):
        cid = lax.axis_index("c")
        sid = lax.axis_index("s")
        wid = cid * NS + sid
        base = wid * S

        pltpu.sync_copy(hist_hbm.at[pl.ds(wid * NP, NP)], hidx_v)
        pltpu.sync_copy(pid_hbm.at[pl.ds(base, S)], pidx_v)

        for kk in range(S // 128):
            pltpu.async_copy(
                table_hbm.at[pidx_v.at[pl.ds(kk * 128, 128)]], prow_v, gsem
            ).wait()
            pltpu.sync_copy(prow_v, prodout_hbm.at[pl.ds(base + kk * 128, 128)])

        def gcopy(p, slot):
            return pltpu.make_async_copy(
                table_hbm.at[hidx_v.at[p]], ring_v.at[slot], gsems.at[slot])

        def wcopy(p, sloc, slot):
            i = base + p * SPS + sloc
            return pltpu.make_async_copy(
                ring_v.at[slot, pl.ds(sloc * HIST, HIST)],
                histout_hbm.at[pl.ds(i * 2 * HPP, HIST)],
                wsems.at[slot])

        for j in range(NFLY):
            gcopy(j, j).start()

        @pl.loop(0, NP)
        def _(p):
            u = lax.rem(p, NBUF)
            gcopy(p, u).wait()
            wcopy(p, 0, u).start()
            wcopy(p, 1, u).start()
            nxt = p + NFLY
            v = lax.rem(nxt, NBUF)

            @pl.when(jnp.logical_and(nxt < NP, nxt >= NBUF))
            def _():
                wcopy(nxt, 0, v).wait()
                wcopy(nxt, 1, v).wait()
                gcopy(nxt, v).start()

            @pl.when(jnp.logical_and(nxt < NP, nxt < NBUF))
            def _():
                gcopy(nxt, v).start()

        for u in range(NBUF):
            wcopy(0, 0, u).wait()
            wcopy(0, 1, u).wait()

    return k(hist2, pids, table)


def _tc_attn_mlp(histp, prod, w2c, b_s, w1a, w1b, b1, w2, b2, nb):
    BS = 256

    def body(h_ref, q_ref, w2c_ref, bs_ref, w1a_ref, w1b_ref, b1_ref,
             w2_ref, b2_ref, o_ref):
        h2d = h_ref[...]
        s2 = jnp.dot(h2d, w2c_ref[...],
                     preferred_element_type=jnp.float32)
        rows = lax.broadcasted_iota(jnp.int32, (BS * HPP, 2), 0)
        valid = lax.rem(rows, HPP) < HP
        e2 = jnp.where(valid, jnp.exp(jnp.tanh(s2 + bs_ref[...])), 0.0)
        e_e = jnp.broadcast_to(e2[:, 0:1], (BS * HPP, PD))
        e_o = jnp.broadcast_to(e2[:, 1:2], (BS * HPP, PD))
        eb = jnp.concatenate([e_e, e_o], axis=1)
        wh = (h2d * eb).reshape(BS, HPP, 2 * PD)
        ssum = jnp.sum(wh, axis=1)
        den = jnp.sum(jnp.sum(e2.reshape(BS, HPP, 2), axis=1),
                      axis=1, keepdims=True)
        pooled = (ssum[:, :PD] + ssum[:, PD:]) / den
        x = jnp.dot(pooled, w1a_ref[...], preferred_element_type=jnp.float32)
        x = x + jnp.dot(q_ref[...], w1b_ref[...],
                        preferred_element_type=jnp.float32)
        x = jnp.maximum(x + b1_ref[...], 0.0)
        y = jnp.dot(x, w2_ref[...], preferred_element_type=jnp.float32)
        o_ref[...] = jax.nn.sigmoid(y + b2_ref[...])

    return pl.pallas_call(
        body,
        grid=(nb // BS,),
        in_specs=[
            pl.BlockSpec((BS * HPP, 2 * PD), lambda i: (i, 0)),
            pl.BlockSpec((BS, PD), lambda i: (i, 0)),
            pl.BlockSpec((2 * PD, 2), lambda i: (0, 0)),
            pl.BlockSpec((1, 1), lambda i: (0, 0)),
            pl.BlockSpec((PD, HID), lambda i: (0, 0)),
            pl.BlockSpec((PD, HID), lambda i: (0, 0)),
            pl.BlockSpec((1, HID), lambda i: (0, 0)),
            pl.BlockSpec((HID, 1), lambda i: (0, 0)),
            pl.BlockSpec((1, 1), lambda i: (0, 0)),
        ],
        out_specs=pl.BlockSpec((BS, 1), lambda i: (i, 0)),
        out_shape=jax.ShapeDtypeStruct((nb, 1), jnp.float32),
    )(histp, prod, w2c, b_s, w1a, w1b, b1, w2, b2)


def kernel(user_ids, product_ids, user_history, user_table, prod_table,
           attn_W, attn_b, mlp1_W, mlp1_b, mlp2_W, mlp2_b):
    hist2 = user_history.astype(jnp.int32).reshape(B // SPS, SPS * HIST)
    pids = product_ids.astype(jnp.int32)
    wcol = attn_W.reshape(PD)
    zz = jnp.zeros((PD,), jnp.float32)
    w2c = jnp.stack([jnp.concatenate([wcol, zz]),
                     jnp.concatenate([zz, wcol])], axis=1)
    NCH = 4
    nb = B // NCH
    outs = []
    for c in range(NCH):
        h2c = lax.slice_in_dim(hist2, c * (nb // SPS), (c + 1) * (nb // SPS))
        pc = lax.slice_in_dim(pids, c * nb, (c + 1) * nb)
        hist_rows, prodrows = _sc_gather(h2c, pc, prod_table, nb)
        histp = hist_rows.reshape(nb * HPP, 2 * PD)
        outs.append(_tc_attn_mlp(histp, prodrows,
                                 w2c,
                                 attn_b.reshape(1, 1),
                                 mlp1_W[:PD], mlp1_W[PD:],
                                 mlp1_b.reshape(1, HID), mlp2_W,
                                 mlp2_b.reshape(1, 1), nb))
    return jnp.concatenate(outs, axis=0)
